# Initial kernel scaffold; baseline (speedup 1.0000x reference)
import jax
import jax.numpy as jnp
from jax.experimental import pallas as pl


def kernel(anchor_emb, pos_emb, neg_emb, neg_batch_indices):
    raise NotImplementedError("write your pallas kernel here")


# speedup vs baseline: 5.3111x; 5.3111x over previous
import dataclasses
import functools

import jax
import jax.numpy as jnp
from jax import lax
from jax.experimental import pallas as pl
from jax.experimental.pallas import tpu as pltpu
from jax.experimental.pallas import tpu_sc as plsc

_B = 16384
_D = 64
_N = 819200
_MARGIN = 0.5

_TW = 80
_NC, _NS = 2, 16
_NW = _NC * _NS
_RPW = _N // _NW
_CH = 128
_NCHUNK = _RPW // _CH


def _prep_body(a_ref, p_ref, out_ref):
    a = a_ref[...]
    p = p_ref[...]
    na2 = jnp.sum(a * a, axis=1, keepdims=True)
    np2 = jnp.sum(p * p, axis=1, keepdims=True)
    dot = jnp.sum(a * p, axis=1, keepdims=True)
    na = jnp.sqrt(na2)
    pos_sim = dot / jnp.maximum(na * jnp.sqrt(np2), 1e-8)
    a_hat = a / jnp.maximum(na, 1e-30)
    out_ref[:, 0:_D] = a_hat
    out_ref[:, _D:_D + 1] = _MARGIN - pos_sim
    out_ref[:, _D + 1:_TW] = jnp.zeros((a.shape[0], _TW - _D - 1), jnp.float32)


_prep = pl.pallas_call(
    _prep_body,
    out_shape=jax.ShapeDtypeStruct((_B, _TW), jnp.float32),
)


def _sc_body(table_hbm, neg_hbm, idx_hbm, sums_hbm, cnts_hbm,
             idx_v, neg_v, row_v, sum_loc, cnt_loc, sem):
    wid = lax.axis_index("s") * _NC + lax.axis_index("c")
    base_w = wid * _RPW

    zeros16 = jnp.zeros((16,), jnp.float32)
    ones16 = jnp.ones((16,), jnp.float32)
    iota16 = lax.iota(jnp.int32, 16)

    @pl.loop(0, _B, step=16)
    def _(i):
        sum_loc[pl.ds(i, 16)] = zeros16
        cnt_loc[pl.ds(i, 16)] = zeros16

    @pl.loop(0, _NCHUNK)
    def _(ci):
        base = base_w + ci * _CH
        pltpu.sync_copy(idx_hbm.at[pl.ds(base, _CH)], idx_v)
        pltpu.sync_copy(neg_hbm.at[pl.ds(base, _CH)], neg_v)
        pltpu.async_copy(table_hbm.at[idx_v], row_v, sem).wait()

        @pl.loop(0, _CH, step=16)
        def _(r0):
            rows = r0 + iota16
            dot = zeros16
            nn = zeros16
            for d in range(_D):
                dcol = jnp.full((16,), d, jnp.int32)
                a_d = plsc.load_gather(row_v, [rows, dcol])
                n_d = plsc.load_gather(neg_v, [rows, dcol])
                dot = dot + a_d * n_d
                nn = nn + n_d * n_d
            c = plsc.load_gather(row_v, [rows, jnp.full((16,), _D, jnp.int32)])
            x = jnp.maximum(nn, 1e-30)
            i0 = plsc.bitcast(x, jnp.int32)
            i0 = jnp.int32(0x5F3759DF) - lax.shift_right_logical(i0, 1)
            y = plsc.bitcast(i0, jnp.float32)
            y = y * (1.5 - 0.5 * x * y * y)
            y = y * (1.5 - 0.5 * x * y * y)
            y = y * (1.5 - 0.5 * x * y * y)
            t = jnp.maximum(c + dot * y, 0.0)
            ivals = idx_v[pl.ds(r0, 16)]
            plsc.addupdate_scatter(sum_loc, [ivals], t)
            plsc.addupdate_scatter(cnt_loc, [ivals], ones16)

    pltpu.sync_copy(sum_loc, sums_hbm.at[wid])
    pltpu.sync_copy(cnt_loc, cnts_hbm.at[wid])


_sc_params = pltpu.CompilerParams()
for _f, _v in (("needs_layout_passes", False), ("use_tc_tiling_on_sc", False)):
    if _f in pltpu.CompilerParams.__dataclass_fields__:
        _sc_params = dataclasses.replace(_sc_params, **{_f: _v})

_sc_main = functools.partial(
    pl.kernel,
    mesh=plsc.VectorSubcoreMesh(core_axis_name="c", subcore_axis_name="s"),
    compiler_params=_sc_params,
    out_type=(jax.ShapeDtypeStruct((_NW, _B), jnp.float32),
              jax.ShapeDtypeStruct((_NW, _B), jnp.float32)),
    scratch_types=[
        pltpu.VMEM((_CH,), jnp.int32),
        pltpu.VMEM((_CH, _D), jnp.float32),
        pltpu.VMEM((_CH, _TW), jnp.float32),
        pltpu.VMEM((_B,), jnp.float32),
        pltpu.VMEM((_B,), jnp.float32),
        pltpu.SemaphoreType.DMA,
    ],
)(_sc_body)


def _final_body(sums_ref, cnts_ref, out_ref):
    seg_sum = jnp.sum(sums_ref[...], axis=0)
    seg_cnt = jnp.sum(cnts_ref[...], axis=0)
    mean = jnp.where(seg_cnt > 0, seg_sum / jnp.maximum(seg_cnt, 1.0), 0.0)
    out_ref[...] = jnp.sum(mean).reshape(1, 1) / _B


_final = pl.pallas_call(
    _final_body,
    out_shape=jax.ShapeDtypeStruct((1, 1), jnp.float32),
)


@jax.jit
def kernel(anchor_emb, pos_emb, neg_emb, neg_batch_indices):
    table = _prep(anchor_emb, pos_emb)
    sums, cnts = _sc_main(table, neg_emb, neg_batch_indices)
    out = _final(sums, cnts)
    return out[0, 0]


# speedup vs baseline: 9.5224x; 1.7929x over previous
import dataclasses
import functools

import jax
import jax.numpy as jnp
from jax import lax
from jax.experimental import pallas as pl
from jax.experimental.pallas import tpu as pltpu
from jax.experimental.pallas import tpu_sc as plsc

_B = 16384
_D = 64
_N = 819200
_MARGIN = 0.5

_TW = 80
_NC, _NS = 2, 16
_NW = _NC * _NS
_RPW = _N // _NW
_CH = 256
_NCHUNK = _RPW // _CH


def _prep_body(a_ref, p_ref, out_ref):
    a = a_ref[...]
    p = p_ref[...]
    na2 = jnp.sum(a * a, axis=1, keepdims=True)
    np2 = jnp.sum(p * p, axis=1, keepdims=True)
    dot = jnp.sum(a * p, axis=1, keepdims=True)
    na = jnp.sqrt(na2)
    pos_sim = dot / jnp.maximum(na * jnp.sqrt(np2), 1e-8)
    a_hat = a / jnp.maximum(na, 1e-30)
    out_ref[:, 0:_D] = a_hat
    out_ref[:, _D:_D + 1] = _MARGIN - pos_sim
    out_ref[:, _D + 1:_TW] = jnp.zeros((a.shape[0], _TW - _D - 1), jnp.float32)


_prep = pl.pallas_call(
    _prep_body,
    out_shape=jax.ShapeDtypeStruct((_B, _TW), jnp.float32),
)


def _sc_body(table_hbm, neg_hbm, idx_hbm, sums_hbm, cnts_hbm,
             idx_v, neg_v, row_v, sum_loc, cnt_loc,
             sem_i0, sem_i1, sem_n0, sem_n1, sem_r0, sem_r1):
    wid = lax.axis_index("s") * _NC + lax.axis_index("c")
    base_w = wid * _RPW
    sem_i = (sem_i0, sem_i1)
    sem_n = (sem_n0, sem_n1)
    sem_r = (sem_r0, sem_r1)

    zeros16 = jnp.zeros((16,), jnp.float32)
    ones16 = jnp.ones((16,), jnp.float32)
    iota16 = lax.iota(jnp.int32, 16)

    @pl.loop(0, _B, step=16)
    def _(i):
        sum_loc[pl.ds(i, 16)] = zeros16
        cnt_loc[pl.ds(i, 16)] = zeros16

    def idx_copy(i, b):
        return pltpu.make_async_copy(
            idx_hbm.at[pl.ds(base_w + i * _CH, _CH)], idx_v.at[b], sem_i[b])

    def neg_copy(i, b):
        return pltpu.make_async_copy(
            neg_hbm.at[pl.ds(base_w + i * _CH, _CH)], neg_v.at[b], sem_n[b])

    def row_copy(b, h):
        return pltpu.make_async_copy(
            table_hbm.at[idx_v.at[b, pl.ds(h * 128, 128)]],
            row_v.at[b, pl.ds(h * 128, 128)],
            sem_r[b])

    def compute(b):
        @pl.loop(0, _CH, step=16)
        def _(r0):
            rows = r0 + iota16
            dot = zeros16
            nn = zeros16
            for d in range(_D):
                dcol = jnp.full((16,), d, jnp.int32)
                a_d = plsc.load_gather(row_v.at[b], [rows, dcol])
                n_d = plsc.load_gather(neg_v.at[b], [rows, dcol])
                dot = dot + a_d * n_d
                nn = nn + n_d * n_d
            c = plsc.load_gather(row_v.at[b],
                                 [rows, jnp.full((16,), _D, jnp.int32)])
            x = jnp.maximum(nn, 1e-30)
            i0 = plsc.bitcast(x, jnp.int32)
            i0 = jnp.int32(0x5F3759DF) - lax.shift_right_logical(i0, 1)
            y = plsc.bitcast(i0, jnp.float32)
            y = y * (1.5 - 0.5 * x * y * y)
            y = y * (1.5 - 0.5 * x * y * y)
            y = y * (1.5 - 0.5 * x * y * y)
            t = jnp.maximum(c + dot * y, 0.0)
            ivals = idx_v[b, pl.ds(r0, 16)]
            plsc.addupdate_scatter(sum_loc, [ivals], t)
            plsc.addupdate_scatter(cnt_loc, [ivals], ones16)

    def half(i, b):
        @pl.when(i + 1 < _NCHUNK)
        def _():
            idx_copy(i + 1, 1 - b).wait()
            neg_copy(i + 1, 1 - b).start()
            row_copy(1 - b, 0).start()
            row_copy(1 - b, 1).start()

        neg_copy(i, b).wait()
        row_copy(b, 0).wait()
        row_copy(b, 1).wait()
        compute(b)

        @pl.when(i + 2 < _NCHUNK)
        def _():
            idx_copy(i + 2, b).start()

    pltpu.sync_copy(idx_hbm.at[pl.ds(base_w, _CH)], idx_v.at[0])
    neg_copy(0, 0).start()
    row_copy(0, 0).start()
    row_copy(0, 1).start()
    idx_copy(1, 1).start()

    @pl.loop(0, _NCHUNK, step=2)
    def _(ci):
        half(ci, 0)
        half(ci + 1, 1)

    pltpu.sync_copy(sum_loc, sums_hbm.at[wid])
    pltpu.sync_copy(cnt_loc, cnts_hbm.at[wid])


_sc_params = pltpu.CompilerParams()
for _f, _v in (("needs_layout_passes", False), ("use_tc_tiling_on_sc", False)):
    if _f in pltpu.CompilerParams.__dataclass_fields__:
        _sc_params = dataclasses.replace(_sc_params, **{_f: _v})

_sc_main = functools.partial(
    pl.kernel,
    mesh=plsc.VectorSubcoreMesh(core_axis_name="c", subcore_axis_name="s"),
    compiler_params=_sc_params,
    out_type=(jax.ShapeDtypeStruct((_NW, _B), jnp.float32),
              jax.ShapeDtypeStruct((_NW, _B), jnp.float32)),
    scratch_types=[
        pltpu.VMEM((2, _CH), jnp.int32),
        pltpu.VMEM((2, _CH, _D), jnp.float32),
        pltpu.VMEM((2, _CH, _TW), jnp.float32),
        pltpu.VMEM((_B,), jnp.float32),
        pltpu.VMEM((_B,), jnp.float32),
        pltpu.SemaphoreType.DMA,
        pltpu.SemaphoreType.DMA,
        pltpu.SemaphoreType.DMA,
        pltpu.SemaphoreType.DMA,
        pltpu.SemaphoreType.DMA,
        pltpu.SemaphoreType.DMA,
    ],
)(_sc_body)


def _final_body(sums_ref, cnts_ref, out_ref):
    seg_sum = jnp.sum(sums_ref[...], axis=0)
    seg_cnt = jnp.sum(cnts_ref[...], axis=0)
    mean = jnp.where(seg_cnt > 0, seg_sum / jnp.maximum(seg_cnt, 1.0), 0.0)
    out_ref[...] = jnp.sum(mean).reshape(1, 1) / _B


_final = pl.pallas_call(
    _final_body,
    out_shape=jax.ShapeDtypeStruct((1, 1), jnp.float32),
)


@jax.jit
def kernel(anchor_emb, pos_emb, neg_emb, neg_batch_indices):
    table = _prep(anchor_emb, pos_emb)
    sums, cnts = _sc_main(table, neg_emb, neg_batch_indices)
    out = _final(sums, cnts)
    return out[0, 0]
